# Initial kernel scaffold; baseline (speedup 1.0000x reference)
import jax
import jax.numpy as jnp
from jax.experimental import pallas as pl


def kernel(inputs, table):
    raise NotImplementedError("write your pallas kernel here")


# speedup vs baseline: 1.8313x; 1.8313x over previous
import functools

import jax
import jax.numpy as jnp
from jax import lax
from jax.experimental import pallas as pl
from jax.experimental.pallas import tpu as pltpu
from jax.experimental.pallas import tpu_sc as plsc


@functools.lru_cache(maxsize=None)
def _make_gather(B, D, chunk):
    info = plsc.get_sparse_core_info()
    nc, ns = info.num_cores, info.num_subcores
    nw = nc * ns
    assert B % nw == 0
    b_per_w = B // nw
    assert b_per_w % chunk == 0
    n_chunks = b_per_w // chunk
    mesh = plsc.VectorSubcoreMesh(core_axis_name="c", subcore_axis_name="s")

    @functools.partial(
        pl.kernel,
        mesh=mesh,
        out_type=jax.ShapeDtypeStruct((B, D), jnp.float32),
        compiler_params=pltpu.CompilerParams(use_tc_tiling_on_sc=False),
        scratch_types=[
            pltpu.VMEM((b_per_w,), jnp.int32),
            pltpu.VMEM((chunk, D), jnp.float32),
            pltpu.SemaphoreType.DMA,
        ],
    )
    def k(idx_hbm, table_hbm, out_hbm, idx_v, rows_v, sem):
        wid = lax.axis_index("s") * nc + lax.axis_index("c")
        base = wid * b_per_w
        pltpu.sync_copy(idx_hbm.at[pl.ds(base, b_per_w)], idx_v)
        for c in range(n_chunks):
            off = c * chunk
            pltpu.async_copy(
                table_hbm.at[idx_v.at[pl.ds(off, chunk)]], rows_v, sem
            ).wait()
            pltpu.sync_copy(rows_v, out_hbm.at[pl.ds(base + off, chunk)])

    return k


def kernel(inputs, table):
    batch, hist = inputs.shape
    dim = table.shape[1]
    idx = inputs.reshape(-1).astype(jnp.int32)
    out = _make_gather(idx.shape[0], dim, 512)(idx, table)
    return out.reshape(batch, hist, dim)


# speedup vs baseline: 1.8667x; 1.0193x over previous
import functools

import jax
import jax.numpy as jnp
from jax import lax
from jax.experimental import pallas as pl
from jax.experimental.pallas import tpu as pltpu
from jax.experimental.pallas import tpu_sc as plsc

_NBUF = 4


@functools.lru_cache(maxsize=None)
def _make_gather(B, D, chunk):
    info = plsc.get_sparse_core_info()
    nc, ns = info.num_cores, info.num_subcores
    nw = nc * ns
    assert B % nw == 0
    b_per_w = B // nw
    assert b_per_w % (chunk * _NBUF) == 0
    n_groups = b_per_w // (chunk * _NBUF)
    mesh = plsc.VectorSubcoreMesh(core_axis_name="c", subcore_axis_name="s")

    @functools.partial(
        pl.kernel,
        mesh=mesh,
        out_type=jax.ShapeDtypeStruct((B, D), jnp.float32),
        compiler_params=pltpu.CompilerParams(use_tc_tiling_on_sc=False),
        scratch_types=[
            pltpu.VMEM((b_per_w,), jnp.int32),
            pltpu.VMEM((_NBUF, chunk, D), jnp.float32),
        ]
        + [pltpu.SemaphoreType.DMA] * (2 * _NBUF),
    )
    def k(idx_hbm, table_hbm, out_hbm, idx_v, rows_v, *sems):
        gsem, wsem = sems[:_NBUF], sems[_NBUF:]
        wid = lax.axis_index("s") * nc + lax.axis_index("c")
        base = wid * b_per_w
        pltpu.sync_copy(idx_hbm.at[pl.ds(base, b_per_w)], idx_v)

        def gather(off, b):
            return pltpu.make_async_copy(
                table_hbm.at[idx_v.at[pl.ds(off, chunk)]], rows_v.at[b], gsem[b]
            )

        def write(off, b):
            return pltpu.make_async_copy(
                rows_v.at[b], out_hbm.at[pl.ds(base + off, chunk)], wsem[b]
            )

        for b in range(_NBUF):
            gather(b * chunk, b).start()

        def body(p, _):
            for b in range(_NBUF):
                off = (p * _NBUF + b) * chunk
                gather(off, b).wait()
                write(off, b).start()
            for b in range(_NBUF):
                off = (p * _NBUF + b) * chunk
                noff = ((p + 1) * _NBUF + b) * chunk
                write(off, b).wait()
                gather(noff, b).start()
            return 0

        lax.fori_loop(0, n_groups - 1, body, 0)

        last = (n_groups - 1) * _NBUF
        for b in range(_NBUF):
            off = (last + b) * chunk
            gather(off, b).wait()
            write(off, b).start()
        for b in range(_NBUF):
            off = (last + b) * chunk
            write(off, b).wait()

    return k


def kernel(inputs, table):
    batch, hist = inputs.shape
    dim = table.shape[1]
    idx = inputs.reshape(-1).astype(jnp.int32)
    out = _make_gather(idx.shape[0], dim, 256)(idx, table)
    return out.reshape(batch, hist, dim)


# speedup vs baseline: 1.8768x; 1.0054x over previous
import functools

import jax
import jax.numpy as jnp
from jax import lax
from jax.experimental import pallas as pl
from jax.experimental.pallas import tpu as pltpu
from jax.experimental.pallas import tpu_sc as plsc

_NBUF = 2


@functools.lru_cache(maxsize=None)
def _make_gather(B, D, chunk):
    info = plsc.get_sparse_core_info()
    nc, ns = info.num_cores, info.num_subcores
    nw = nc * ns
    assert B % nw == 0
    b_per_w = B // nw
    assert b_per_w % (chunk * _NBUF) == 0
    n_groups = b_per_w // (chunk * _NBUF)
    mesh = plsc.VectorSubcoreMesh(core_axis_name="c", subcore_axis_name="s")

    @functools.partial(
        pl.kernel,
        mesh=mesh,
        out_type=jax.ShapeDtypeStruct((B, D), jnp.float32),
        compiler_params=pltpu.CompilerParams(use_tc_tiling_on_sc=False),
        scratch_types=[
            pltpu.VMEM((b_per_w,), jnp.int32),
            pltpu.VMEM((_NBUF, chunk, D), jnp.float32),
        ]
        + [pltpu.SemaphoreType.DMA] * (2 * _NBUF),
    )
    def k(idx_hbm, table_hbm, out_hbm, idx_v, rows_v, *sems):
        gsem, wsem = sems[:_NBUF], sems[_NBUF:]
        wid = lax.axis_index("s") * nc + lax.axis_index("c")
        base = wid * b_per_w
        pltpu.sync_copy(idx_hbm.at[pl.ds(base, b_per_w)], idx_v)

        def gather(off, b):
            return pltpu.make_async_copy(
                table_hbm.at[idx_v.at[pl.ds(off, chunk)]], rows_v.at[b], gsem[b]
            )

        def write(off, b):
            return pltpu.make_async_copy(
                rows_v.at[b], out_hbm.at[pl.ds(base + off, chunk)], wsem[b]
            )

        for b in range(_NBUF):
            gather(b * chunk, b).start()

        def body(p, _):
            for b in range(_NBUF):
                off = (p * _NBUF + b) * chunk
                gather(off, b).wait()
                write(off, b).start()
            for b in range(_NBUF):
                off = (p * _NBUF + b) * chunk
                noff = ((p + 1) * _NBUF + b) * chunk
                write(off, b).wait()
                gather(noff, b).start()
            return 0

        lax.fori_loop(0, n_groups - 1, body, 0)

        last = (n_groups - 1) * _NBUF
        for b in range(_NBUF):
            off = (last + b) * chunk
            gather(off, b).wait()
            write(off, b).start()
        for b in range(_NBUF):
            off = (last + b) * chunk
            write(off, b).wait()

    return k


def kernel(inputs, table):
    batch, hist = inputs.shape
    dim = table.shape[1]
    idx = inputs.reshape(-1).astype(jnp.int32)
    out = _make_gather(idx.shape[0], dim, 512)(idx, table)
    return out.reshape(batch, hist, dim)
